# Initial kernel scaffold; baseline (speedup 1.0000x reference)
import jax
import jax.numpy as jnp
from jax.experimental import pallas as pl


def kernel(node_features, edge_index, W1, b1, W2, b2, gate_W, gate_b, is_training):
    raise NotImplementedError("write your pallas kernel here")


# speedup vs baseline: 3.1673x; 3.1673x over previous
import jax
import jax.numpy as jnp
from jax import lax
from jax.experimental import pallas as pl
from jax.experimental.pallas import tpu as pltpu
from jax.experimental.pallas import tpu_sc as plsc

N = 10000
E = 320000
D = 128
K = 20

NC = 2
NS = 16
NW = NC * NS
L = 16

N_PAD = 10240
E_PAD = 327680
CW = 128
EC = E_PAD // NW // CW
ECD = E_PAD // NS // CW
ROWS_PT = N_PAD // NS
ROWS_PW = N_PAD // NW

_MESH = plsc.VectorSubcoreMesh(core_axis_name="c", subcore_axis_name="s")


def _worker_id():
    return lax.axis_index("c") * NS + lax.axis_index("s")


def _fill_rows(ref, rows, value):
    vec = jnp.full((L,), value, jnp.float32)
    width = ref.shape[1]

    def body(i, _):
        for j in range(width // L):
            ref[i, pl.ds(j * L, L)] = vec
        return 0

    lax.fori_loop(0, rows, body, 0)


def _iota_row(idx_ref, start):
    ii = lax.iota(jnp.int32, L)
    for k in range(CW // L):
        idx_ref[0, pl.ds(k * L, L)] = ii + (start + k * L)


def _zero_my_acc_rows(acc, idx_ref, zrows, base, sem):
    def z(i, _):
        _iota_row(idx_ref, base + i * CW)
        pltpu.async_copy(zrows, acc.at[idx_ref.at[0]], sem).wait()
        return 0
    lax.fori_loop(0, ROWS_PT // CW, z, 0)


def _emit_my_acc_rows(acc, idx_ref, rows_v, base, out_ref_slicer, sem):
    def rd(i, _):
        off = base + i * CW
        _iota_row(idx_ref, off)
        pltpu.async_copy(acc.at[idx_ref.at[0]], rows_v, sem).wait()
        pltpu.sync_copy(rows_v, out_ref_slicer(off))
        return 0
    lax.fori_loop(0, ROWS_PT // CW, rd, 0)


def _sc_degrees(ek_t):
    def body(ek_hbm, deg_hbm, idx_v, rows_v, acc, sem):
        c = lax.axis_index("c")
        s = lax.axis_index("s")
        base = s * ROWS_PT

        _fill_rows(rows_v, CW, 0.0)
        _zero_my_acc_rows(acc, idx_v, rows_v, base, sem)
        plsc.subcore_barrier()
        _fill_rows(rows_v, CW, 1.0)

        pltpu.sync_copy(ek_hbm.at[c, s], idx_v)

        def chunk(ch, _):
            pltpu.async_copy(rows_v, acc.at[idx_v.at[ch]], sem,
                             add=True).wait()
            return 0
        lax.fori_loop(0, ECD, chunk, 0)
        plsc.subcore_barrier()

        _emit_my_acc_rows(acc, idx_v, rows_v, base,
                          lambda off: deg_hbm.at[c, pl.ds(off, CW)], sem)

    f = pl.kernel(
        body,
        out_type=jax.ShapeDtypeStruct((NC, N_PAD, D), jnp.float32),
        mesh=_MESH,
        scratch_types=[
            pltpu.VMEM((ECD, CW), jnp.int32),
            pltpu.VMEM((CW, D), jnp.float32),
            pltpu.VMEM_SHARED((N_PAD, D), jnp.float32),
            pltpu.SemaphoreType.DMA,
        ],
    )
    return f(ek_t)


def _sc_round(y, src_t, dst_t):
    def body(y_hbm, src_hbm, dst_hbm, part_hbm, src_v, dst_v, rows_v, acc,
             gsem, ssem):
        c = lax.axis_index("c")
        s = lax.axis_index("s")
        w = c * NS + s
        base = s * ROWS_PT

        _fill_rows(rows_v, CW, 0.0)
        _zero_my_acc_rows(acc, src_v, rows_v, base, ssem)
        plsc.subcore_barrier()

        pltpu.sync_copy(src_hbm.at[w], src_v)
        pltpu.sync_copy(dst_hbm.at[w], dst_v)

        def chunk(ch, _):
            pltpu.async_copy(y_hbm.at[src_v.at[ch]], rows_v, gsem).wait()
            pltpu.async_copy(rows_v, acc.at[dst_v.at[ch]], ssem,
                             add=True).wait()
            return 0
        lax.fori_loop(0, EC, chunk, 0)
        plsc.subcore_barrier()

        _emit_my_acc_rows(acc, src_v, rows_v, base,
                          lambda off: part_hbm.at[c, pl.ds(off, CW)], gsem)

    f = pl.kernel(
        body,
        out_type=jax.ShapeDtypeStruct((NC, N_PAD, D), jnp.float32),
        mesh=_MESH,
        scratch_types=[
            pltpu.VMEM((EC, CW), jnp.int32),
            pltpu.VMEM((EC, CW), jnp.int32),
            pltpu.VMEM((CW, D), jnp.float32),
            pltpu.VMEM_SHARED((N_PAD, D), jnp.float32),
            pltpu.SemaphoreType.DMA,
            pltpu.SemaphoreType.DMA,
        ],
    )
    return f(y, src_t, dst_t)


_CCH = 64


def _sc_combine(part, rin, rout):
    def body(part_hbm, rin_hbm, rout_hbm, x_hbm, y_hbm,
             rin_v, rout_v, a_v, b_v, x_v, y_v, sem):
        w = _worker_id()
        base = w * ROWS_PW
        pltpu.sync_copy(rin_hbm.at[pl.ds(base, ROWS_PW)], rin_v)
        pltpu.sync_copy(rout_hbm.at[pl.ds(base, ROWS_PW)], rout_v)

        def chunk(ch, _):
            off = base + ch * _CCH
            pltpu.sync_copy(part_hbm.at[0, pl.ds(off, _CCH)], a_v)
            pltpu.sync_copy(part_hbm.at[1, pl.ds(off, _CCH)], b_v)

            def row(r, _):
                lr = ch * _CCH + r
                si = rin_v[lr, pl.ds(0, L)]
                so = rout_v[lr, pl.ds(0, L)]
                for j in range(D // L):
                    sl = pl.ds(j * L, L)
                    xv = (a_v[r, sl] + b_v[r, sl]) * si
                    x_v[r, sl] = xv
                    y_v[r, sl] = xv * so
                return 0
            lax.fori_loop(0, _CCH, row, 0)
            pltpu.sync_copy(x_v, x_hbm.at[pl.ds(off, _CCH)])
            pltpu.sync_copy(y_v, y_hbm.at[pl.ds(off, _CCH)])
            return 0
        lax.fori_loop(0, ROWS_PW // _CCH, chunk, 0)

    f = pl.kernel(
        body,
        out_type=(jax.ShapeDtypeStruct((N_PAD, D), jnp.float32),
                  jax.ShapeDtypeStruct((N_PAD, D), jnp.float32)),
        mesh=_MESH,
        scratch_types=[
            pltpu.VMEM((ROWS_PW, L), jnp.float32),
            pltpu.VMEM((ROWS_PW, L), jnp.float32),
            pltpu.VMEM((_CCH, D), jnp.float32),
            pltpu.VMEM((_CCH, D), jnp.float32),
            pltpu.VMEM((_CCH, D), jnp.float32),
            pltpu.VMEM((_CCH, D), jnp.float32),
            pltpu.SemaphoreType.DMA,
        ],
    )
    return f(part, rin, rout)


def _sc_scale0(x0, rout):
    def body(x_hbm, rout_hbm, y_hbm, rout_v, a_v, y_v, sem):
        w = _worker_id()
        base = w * ROWS_PW
        pltpu.sync_copy(rout_hbm.at[pl.ds(base, ROWS_PW)], rout_v)

        def chunk(ch, _):
            off = base + ch * _CCH
            pltpu.sync_copy(x_hbm.at[pl.ds(off, _CCH)], a_v)

            def row(r, _):
                so = rout_v[ch * _CCH + r, pl.ds(0, L)]
                for j in range(D // L):
                    sl = pl.ds(j * L, L)
                    y_v[r, sl] = a_v[r, sl] * so
                return 0
            lax.fori_loop(0, _CCH, row, 0)
            pltpu.sync_copy(y_v, y_hbm.at[pl.ds(off, _CCH)])
            return 0
        lax.fori_loop(0, ROWS_PW // _CCH, chunk, 0)

    f = pl.kernel(
        body,
        out_type=jax.ShapeDtypeStruct((N_PAD, D), jnp.float32),
        mesh=_MESH,
        scratch_types=[
            pltpu.VMEM((ROWS_PW, L), jnp.float32),
            pltpu.VMEM((_CCH, D), jnp.float32),
            pltpu.VMEM((_CCH, D), jnp.float32),
            pltpu.SemaphoreType.DMA,
        ],
    )
    return f(x0, rout)


_MB = 512


def _tc_mlp(nf_p, W1, b1, W2, b2):
    def body(x_ref, w1_ref, b1_ref, w2_ref, b2_ref, o_ref):
        h = jnp.maximum(
            jnp.dot(x_ref[...], w1_ref[...],
                    preferred_element_type=jnp.float32) + b1_ref[...], 0.0)
        o_ref[...] = jnp.dot(h, w2_ref[...],
                             preferred_element_type=jnp.float32) + b2_ref[...]

    return pl.pallas_call(
        body,
        grid=(N_PAD // _MB,),
        in_specs=[
            pl.BlockSpec((_MB, D), lambda i: (i, 0)),
            pl.BlockSpec((D, D), lambda i: (0, 0)),
            pl.BlockSpec((1, D), lambda i: (0, 0)),
            pl.BlockSpec((D, D), lambda i: (0, 0)),
            pl.BlockSpec((1, D), lambda i: (0, 0)),
        ],
        out_specs=pl.BlockSpec((_MB, D), lambda i: (i, 0)),
        out_shape=jax.ShapeDtypeStruct((N_PAD, D), jnp.float32),
    )(nf_p, W1, b1.reshape(1, D), W2, b2.reshape(1, D))


def _tc_rsqrt(deg):
    def body(d_ref, ro_ref, ri_ref):
        ro_ref[...] = lax.rsqrt(jnp.maximum(d_ref[0], 1.0))
        ri_ref[...] = lax.rsqrt(jnp.maximum(d_ref[1], 1.0))

    return pl.pallas_call(
        body,
        out_shape=(jax.ShapeDtypeStruct((N_PAD, L), jnp.float32),
                   jax.ShapeDtypeStruct((N_PAD, L), jnp.float32)),
    )(deg)


_GB = 400


def _tc_gate(xs, gate_W, gate_b):
    n_in = len(xs)

    def body(*refs):
        x_refs = refs[:n_in]
        gw_ref, gb_ref, o_ref = refs[n_in], refs[n_in + 1], refs[n_in + 2]
        gw = gw_ref[...]
        gb = gb_ref[0, 0]
        acc = jnp.zeros((_GB, D), jnp.float32)
        for xr in x_refs:
            xk = xr[...]
            z = jnp.dot(xk, gw, preferred_element_type=jnp.float32) + gb
            acc = acc + xk * jax.nn.sigmoid(z)
        o_ref[...] = acc

    return pl.pallas_call(
        body,
        grid=(N // _GB,),
        in_specs=[pl.BlockSpec((_GB, D), lambda i: (i, 0))] * n_in
        + [pl.BlockSpec((D, 1), lambda i: (0, 0)),
           pl.BlockSpec(memory_space=pltpu.SMEM)],
        out_specs=pl.BlockSpec((_GB, D), lambda i: (i, 0)),
        out_shape=jax.ShapeDtypeStruct((N, D), jnp.float32),
    )(*xs, gate_W, gate_b.reshape(1, 1))


def kernel(node_features, edge_index, W1, b1, W2, b2, gate_W, gate_b,
           is_training=True):
    src = edge_index[0]
    dst = edge_index[1]
    pad = jnp.full((E_PAD - E,), N_PAD - 1, jnp.int32)
    src_f = jnp.concatenate([src, pad])
    dst_f = jnp.concatenate([dst, pad])
    src_t = src_f.reshape(NW, EC, CW)
    dst_t = dst_f.reshape(NW, EC, CW)
    ek_t = jnp.stack([src_f.reshape(NS, ECD, CW),
                      dst_f.reshape(NS, ECD, CW)])

    nf_p = jnp.pad(node_features, ((0, N_PAD - N), (0, 0)))

    x0 = _tc_mlp(nf_p, W1, b1, W2, b2)
    deg = _sc_degrees(ek_t)
    rout, rin = _tc_rsqrt(deg[:, :, :L])

    xs = [x0]
    y = _sc_scale0(x0, rout)
    for _ in range(K):
        part = _sc_round(y, src_t, dst_t)
        xk, y = _sc_combine(part, rin, rout)
        xs.append(xk)

    return _tc_gate(xs, gate_W, gate_b)


# speedup vs baseline: 3.4855x; 1.1005x over previous
import jax
import jax.numpy as jnp
from jax import lax
from jax.experimental import pallas as pl
from jax.experimental.pallas import tpu as pltpu
from jax.experimental.pallas import tpu_sc as plsc

N = 10000
E = 320000
D = 128
K = 20

NC = 2
NS = 16
NW = NC * NS
L = 16

N_PAD = 10240
E_PAD = 327680
CW = 128
EC = E_PAD // NW // CW
PH = EC // 2
ECD = E_PAD // NS // CW
ROWS_PT = N_PAD // NS
ROWS_PW = N_PAD // NW

_MESH = plsc.VectorSubcoreMesh(core_axis_name="c", subcore_axis_name="s")


def _worker_id():
    return lax.axis_index("c") * NS + lax.axis_index("s")


def _fill_rows(ref, rows, value):
    vec = jnp.full((L,), value, jnp.float32)
    width = ref.shape[1]

    def body(i, _):
        for j in range(width // L):
            ref[i, pl.ds(j * L, L)] = vec
        return 0

    lax.fori_loop(0, rows, body, 0)


def _iota_row(idx_ref, start, width):
    ii = lax.iota(jnp.int32, L)
    for k in range(width // L):
        idx_ref[0, pl.ds(k * L, L)] = ii + (start + k * L)


def _zero_my_acc_rows(acc, idx_ref, zrows, base, sem):
    width = zrows.shape[0]

    def z(i, _):
        _iota_row(idx_ref, base + i * width, width)
        pltpu.async_copy(zrows, acc.at[idx_ref.at[0]], sem).wait()
        return 0
    lax.fori_loop(0, ROWS_PT // width, z, 0)


def _emit_my_acc_rows(acc, idx_ref, rows_v, base, out_ref_slicer, sem):
    width = rows_v.shape[0]

    def rd(i, _):
        off = base + i * width
        _iota_row(idx_ref, off, width)
        pltpu.async_copy(acc.at[idx_ref.at[0]], rows_v, sem).wait()
        pltpu.sync_copy(rows_v, out_ref_slicer(off))
        return 0
    lax.fori_loop(0, ROWS_PT // width, rd, 0)


def _sc_degrees(ek_t):
    def body(ek_hbm, deg_hbm, idx_v, rows_v, acc, sem):
        c = lax.axis_index("c")
        s = lax.axis_index("s")
        base = s * ROWS_PT

        _fill_rows(rows_v, CW, 0.0)
        _zero_my_acc_rows(acc, idx_v, rows_v, base, sem)
        plsc.subcore_barrier()
        _fill_rows(rows_v, CW, 1.0)

        pltpu.sync_copy(ek_hbm.at[c, s], idx_v)

        def chunk(ch, _):
            pltpu.async_copy(rows_v, acc.at[idx_v.at[ch]], sem,
                             add=True).wait()
            return 0
        lax.fori_loop(0, ECD, chunk, 0)
        plsc.subcore_barrier()

        _emit_my_acc_rows(acc, idx_v, rows_v, base,
                          lambda off: deg_hbm.at[c, pl.ds(off, CW)], sem)

    f = pl.kernel(
        body,
        out_type=jax.ShapeDtypeStruct((NC, N_PAD, D), jnp.float32),
        mesh=_MESH,
        scratch_types=[
            pltpu.VMEM((ECD, CW), jnp.int32),
            pltpu.VMEM((CW, D), jnp.float32),
            pltpu.VMEM_SHARED((N_PAD, D), jnp.float32),
            pltpu.SemaphoreType.DMA,
        ],
    )
    return f(ek_t)


def _sc_round(y, src_t, dst_t):
    def body(y_hbm, src_hbm, dst_hbm, part_hbm, src_v, dst_v, r0, r1, acc,
             gs0, gs1, ssem):
        c = lax.axis_index("c")
        s = lax.axis_index("s")
        w = c * NS + s
        base = s * ROWS_PT

        _fill_rows(r0, CW, 0.0)
        _zero_my_acc_rows(acc, src_v, r0, base, ssem)
        plsc.subcore_barrier()

        def gather(ch, buf, sem):
            return pltpu.async_copy(y_hbm.at[src_v.at[ch]], buf, sem)

        def scatter(ch, buf):
            pltpu.async_copy(buf, acc.at[dst_v.at[ch]], ssem,
                             add=True).wait()

        def phase(p):
            pltpu.sync_copy(src_hbm.at[w, pl.ds(p * PH, PH)], src_v)
            pltpu.sync_copy(dst_hbm.at[w, pl.ds(p * PH, PH)], dst_v)
            gather(0, r0, gs0)

            def step(t, _):
                e = 2 * t
                pltpu.make_async_copy(y_hbm.at[src_v.at[e]], r0, gs0).wait()
                gather(e + 1, r1, gs1)
                scatter(e, r0)
                gather(e + 2, r0, gs0)
                pltpu.make_async_copy(y_hbm.at[src_v.at[e + 1]], r1,
                                      gs1).wait()
                scatter(e + 1, r1)
                return 0
            lax.fori_loop(0, PH // 2 - 1, step, 0)

            e = PH - 2
            pltpu.make_async_copy(y_hbm.at[src_v.at[e]], r0, gs0).wait()
            gather(e + 1, r1, gs1)
            scatter(e, r0)
            pltpu.make_async_copy(y_hbm.at[src_v.at[e + 1]], r1, gs1).wait()
            scatter(e + 1, r1)

        phase(0)
        phase(1)
        plsc.subcore_barrier()

        _emit_my_acc_rows(acc, src_v, r0, base,
                          lambda off: part_hbm.at[c, pl.ds(off, CW)], gs0)

    f = pl.kernel(
        body,
        out_type=jax.ShapeDtypeStruct((NC, N_PAD, D), jnp.float32),
        mesh=_MESH,
        scratch_types=[
            pltpu.VMEM((PH, CW), jnp.int32),
            pltpu.VMEM((PH, CW), jnp.int32),
            pltpu.VMEM((CW, D), jnp.float32),
            pltpu.VMEM((CW, D), jnp.float32),
            pltpu.VMEM_SHARED((N_PAD, D), jnp.float32),
            pltpu.SemaphoreType.DMA,
            pltpu.SemaphoreType.DMA,
            pltpu.SemaphoreType.DMA,
        ],
    )
    return f(y, src_t, dst_t)


_CCH = 64


def _sc_combine(part, rin, rout):
    def body(part_hbm, rin_hbm, rout_hbm, x_hbm, y_hbm,
             rin_v, rout_v, a_v, b_v, x_v, y_v, sem):
        w = _worker_id()
        base = w * ROWS_PW
        pltpu.sync_copy(rin_hbm.at[pl.ds(base, ROWS_PW)], rin_v)
        pltpu.sync_copy(rout_hbm.at[pl.ds(base, ROWS_PW)], rout_v)

        def chunk(ch, _):
            off = base + ch * _CCH
            pltpu.sync_copy(part_hbm.at[0, pl.ds(off, _CCH)], a_v)
            pltpu.sync_copy(part_hbm.at[1, pl.ds(off, _CCH)], b_v)

            def row(r, _):
                lr = ch * _CCH + r
                si = rin_v[lr, pl.ds(0, L)]
                so = rout_v[lr, pl.ds(0, L)]
                for j in range(D // L):
                    sl = pl.ds(j * L, L)
                    xv = (a_v[r, sl] + b_v[r, sl]) * si
                    x_v[r, sl] = xv
                    y_v[r, sl] = xv * so
                return 0
            lax.fori_loop(0, _CCH, row, 0)
            pltpu.sync_copy(x_v, x_hbm.at[pl.ds(off, _CCH)])
            pltpu.sync_copy(y_v, y_hbm.at[pl.ds(off, _CCH)])
            return 0
        lax.fori_loop(0, ROWS_PW // _CCH, chunk, 0)

    f = pl.kernel(
        body,
        out_type=(jax.ShapeDtypeStruct((N_PAD, D), jnp.float32),
                  jax.ShapeDtypeStruct((N_PAD, D), jnp.float32)),
        mesh=_MESH,
        scratch_types=[
            pltpu.VMEM((ROWS_PW, L), jnp.float32),
            pltpu.VMEM((ROWS_PW, L), jnp.float32),
            pltpu.VMEM((_CCH, D), jnp.float32),
            pltpu.VMEM((_CCH, D), jnp.float32),
            pltpu.VMEM((_CCH, D), jnp.float32),
            pltpu.VMEM((_CCH, D), jnp.float32),
            pltpu.SemaphoreType.DMA,
        ],
    )
    return f(part, rin, rout)


def _sc_scale0(x0, rout):
    def body(x_hbm, rout_hbm, y_hbm, rout_v, a_v, y_v, sem):
        w = _worker_id()
        base = w * ROWS_PW
        pltpu.sync_copy(rout_hbm.at[pl.ds(base, ROWS_PW)], rout_v)

        def chunk(ch, _):
            off = base + ch * _CCH
            pltpu.sync_copy(x_hbm.at[pl.ds(off, _CCH)], a_v)

            def row(r, _):
                so = rout_v[ch * _CCH + r, pl.ds(0, L)]
                for j in range(D // L):
                    sl = pl.ds(j * L, L)
                    y_v[r, sl] = a_v[r, sl] * so
                return 0
            lax.fori_loop(0, _CCH, row, 0)
            pltpu.sync_copy(y_v, y_hbm.at[pl.ds(off, _CCH)])
            return 0
        lax.fori_loop(0, ROWS_PW // _CCH, chunk, 0)

    f = pl.kernel(
        body,
        out_type=jax.ShapeDtypeStruct((N_PAD, D), jnp.float32),
        mesh=_MESH,
        scratch_types=[
            pltpu.VMEM((ROWS_PW, L), jnp.float32),
            pltpu.VMEM((_CCH, D), jnp.float32),
            pltpu.VMEM((_CCH, D), jnp.float32),
            pltpu.SemaphoreType.DMA,
        ],
    )
    return f(x0, rout)


_MB = 512


def _tc_mlp(nf_p, W1, b1, W2, b2):
    def body(x_ref, w1_ref, b1_ref, w2_ref, b2_ref, o_ref):
        h = jnp.maximum(
            jnp.dot(x_ref[...], w1_ref[...],
                    preferred_element_type=jnp.float32) + b1_ref[...], 0.0)
        o_ref[...] = jnp.dot(h, w2_ref[...],
                             preferred_element_type=jnp.float32) + b2_ref[...]

    return pl.pallas_call(
        body,
        grid=(N_PAD // _MB,),
        in_specs=[
            pl.BlockSpec((_MB, D), lambda i: (i, 0)),
            pl.BlockSpec((D, D), lambda i: (0, 0)),
            pl.BlockSpec((1, D), lambda i: (0, 0)),
            pl.BlockSpec((D, D), lambda i: (0, 0)),
            pl.BlockSpec((1, D), lambda i: (0, 0)),
        ],
        out_specs=pl.BlockSpec((_MB, D), lambda i: (i, 0)),
        out_shape=jax.ShapeDtypeStruct((N_PAD, D), jnp.float32),
    )(nf_p, W1, b1.reshape(1, D), W2, b2.reshape(1, D))


def _tc_rsqrt(deg):
    def body(d_ref, ro_ref, ri_ref):
        ro_ref[...] = lax.rsqrt(jnp.maximum(d_ref[0], 1.0))
        ri_ref[...] = lax.rsqrt(jnp.maximum(d_ref[1], 1.0))

    return pl.pallas_call(
        body,
        out_shape=(jax.ShapeDtypeStruct((N_PAD, L), jnp.float32),
                   jax.ShapeDtypeStruct((N_PAD, L), jnp.float32)),
    )(deg)


_GB = 400


def _tc_gate(xs, gate_W, gate_b):
    n_in = len(xs)

    def body(*refs):
        x_refs = refs[:n_in]
        gw_ref, gb_ref, o_ref = refs[n_in], refs[n_in + 1], refs[n_in + 2]
        gw = gw_ref[...]
        gb = gb_ref[0, 0]
        acc = jnp.zeros((_GB, D), jnp.float32)
        for xr in x_refs:
            xk = xr[...]
            z = jnp.dot(xk, gw, preferred_element_type=jnp.float32) + gb
            acc = acc + xk * jax.nn.sigmoid(z)
        o_ref[...] = acc

    return pl.pallas_call(
        body,
        grid=(N // _GB,),
        in_specs=[pl.BlockSpec((_GB, D), lambda i: (i, 0))] * n_in
        + [pl.BlockSpec((D, 1), lambda i: (0, 0)),
           pl.BlockSpec(memory_space=pltpu.SMEM)],
        out_specs=pl.BlockSpec((_GB, D), lambda i: (i, 0)),
        out_shape=jax.ShapeDtypeStruct((N, D), jnp.float32),
    )(*xs, gate_W, gate_b.reshape(1, 1))


def kernel(node_features, edge_index, W1, b1, W2, b2, gate_W, gate_b,
           is_training=True):
    src = edge_index[0]
    dst = edge_index[1]
    pad = jnp.full((E_PAD - E,), N_PAD - 1, jnp.int32)
    src_f = jnp.concatenate([src, pad])
    dst_f = jnp.concatenate([dst, pad])
    src_t = src_f.reshape(NW, EC, CW)
    dst_t = dst_f.reshape(NW, EC, CW)
    ek_t = jnp.stack([src_f.reshape(NS, ECD, CW),
                      dst_f.reshape(NS, ECD, CW)])

    nf_p = jnp.pad(node_features, ((0, N_PAD - N), (0, 0)))

    x0 = _tc_mlp(nf_p, W1, b1, W2, b2)
    deg = _sc_degrees(ek_t)
    rout, rin = _tc_rsqrt(deg[:, :, :L])

    xs = [x0]
    y = _sc_scale0(x0, rout)
    for _ in range(K):
        part = _sc_round(y, src_t, dst_t)
        xk, y = _sc_combine(part, rin, rout)
        xs.append(xk)

    return _tc_gate(xs, gate_W, gate_b)


# speedup vs baseline: 3.4945x; 1.0026x over previous
import jax
import jax.numpy as jnp
from jax import lax
from jax.experimental import pallas as pl
from jax.experimental.pallas import tpu as pltpu
from jax.experimental.pallas import tpu_sc as plsc

N = 10000
E = 320000
D = 128
K = 20

NC = 2
NS = 16
NW = NC * NS
L = 16

N_PAD = 10240
E_PAD = 327680
CW = 128
EC = E_PAD // NW // CW
PH = EC // 2
ECD = E_PAD // NS // CW
ROWS_PT = N_PAD // NS
ROWS_PW = N_PAD // NW

_MESH = plsc.VectorSubcoreMesh(core_axis_name="c", subcore_axis_name="s")


def _worker_id():
    return lax.axis_index("c") * NS + lax.axis_index("s")


def _fill_rows(ref, rows, value):
    vec = jnp.full((L,), value, jnp.float32)
    width = ref.shape[1]

    def body(i, _):
        for j in range(width // L):
            ref[i, pl.ds(j * L, L)] = vec
        return 0

    lax.fori_loop(0, rows, body, 0)


def _iota_row(idx_ref, start, width, row=0):
    ii = lax.iota(jnp.int32, L)
    for k in range(width // L):
        idx_ref[row, pl.ds(k * L, L)] = ii + (start + k * L)


def _zero_my_acc_rows(acc, idx_ref, zrows, base, sem):
    width = zrows.shape[0]

    def z(i, _):
        _iota_row(idx_ref, base + i * width, width)
        pltpu.async_copy(zrows, acc.at[idx_ref.at[0]], sem).wait()
        return 0
    lax.fori_loop(0, ROWS_PT // width, z, 0)


def _emit_my_acc_rows(acc, idx_ref, rows_v, base, out_ref_slicer, sem):
    width = rows_v.shape[0]

    def rd(i, _):
        off = base + i * width
        _iota_row(idx_ref, off, width)
        pltpu.async_copy(acc.at[idx_ref.at[0]], rows_v, sem).wait()
        pltpu.sync_copy(rows_v, out_ref_slicer(off))
        return 0
    lax.fori_loop(0, ROWS_PT // width, rd, 0)


def _sc_degrees(ek_t):
    def body(ek_hbm, deg_hbm, idx_v, rows_v, acc, sem):
        c = lax.axis_index("c")
        s = lax.axis_index("s")
        base = s * ROWS_PT

        _fill_rows(rows_v, CW, 0.0)
        _zero_my_acc_rows(acc, idx_v, rows_v, base, sem)
        plsc.subcore_barrier()
        _fill_rows(rows_v, CW, 1.0)

        pltpu.sync_copy(ek_hbm.at[c, s], idx_v)

        def chunk(ch, _):
            pltpu.async_copy(rows_v, acc.at[idx_v.at[ch]], sem,
                             add=True).wait()
            return 0
        lax.fori_loop(0, ECD, chunk, 0)
        plsc.subcore_barrier()

        _emit_my_acc_rows(acc, idx_v, rows_v, base,
                          lambda off: deg_hbm.at[c, pl.ds(off, CW)], sem)

    f = pl.kernel(
        body,
        out_type=jax.ShapeDtypeStruct((NC, N_PAD, D), jnp.float32),
        mesh=_MESH,
        scratch_types=[
            pltpu.VMEM((ECD, CW), jnp.int32),
            pltpu.VMEM((CW, D), jnp.float32),
            pltpu.VMEM_SHARED((N_PAD, D), jnp.float32),
            pltpu.SemaphoreType.DMA,
        ],
    )
    return f(ek_t)


def _sc_round(y, src_t, dst_t):
    def body(y_hbm, src_hbm, dst_hbm, part_hbm, src_v, dst_v, r0, r1, acc,
             gs0, gs1, ssem):
        c = lax.axis_index("c")
        s = lax.axis_index("s")
        w = c * NS + s
        base = s * ROWS_PT

        _fill_rows(r0, CW, 0.0)
        for i in range(ROWS_PT // CW):
            _iota_row(src_v, base + i * CW, CW, row=i)
        for i in range(ROWS_PT // CW):
            pltpu.async_copy(r0, acc.at[src_v.at[i]], ssem)
        for i in range(ROWS_PT // CW):
            pltpu.make_async_copy(r0, acc.at[src_v.at[i]], ssem).wait()
        plsc.subcore_barrier()

        def gather(ch, buf, sem):
            return pltpu.async_copy(y_hbm.at[src_v.at[ch]], buf, sem)

        def scatter(ch, buf):
            pltpu.async_copy(buf, acc.at[dst_v.at[ch]], ssem,
                             add=True).wait()

        def phase(p):
            pltpu.sync_copy(src_hbm.at[w, pl.ds(p * PH, PH)], src_v)
            pltpu.sync_copy(dst_hbm.at[w, pl.ds(p * PH, PH)], dst_v)
            gather(0, r0, gs0)

            def step(t, _):
                e = 2 * t
                pltpu.make_async_copy(y_hbm.at[src_v.at[e]], r0, gs0).wait()
                gather(e + 1, r1, gs1)
                scatter(e, r0)
                gather(e + 2, r0, gs0)
                pltpu.make_async_copy(y_hbm.at[src_v.at[e + 1]], r1,
                                      gs1).wait()
                scatter(e + 1, r1)
                return 0
            lax.fori_loop(0, PH // 2 - 1, step, 0)

            e = PH - 2
            pltpu.make_async_copy(y_hbm.at[src_v.at[e]], r0, gs0).wait()
            gather(e + 1, r1, gs1)
            scatter(e, r0)
            pltpu.make_async_copy(y_hbm.at[src_v.at[e + 1]], r1, gs1).wait()
            scatter(e + 1, r1)

        phase(0)
        phase(1)
        plsc.subcore_barrier()

        for i in range(ROWS_PT // CW):
            _iota_row(src_v, base + i * CW, CW, row=i)
        bufs = (r0, r1)
        for i in range(ROWS_PT // CW):
            b = bufs[i % 2]
            if i >= 2:
                off_p = base + (i - 2) * CW
                pltpu.make_async_copy(
                    b, part_hbm.at[c, pl.ds(off_p, CW)], gs1).wait()
            pltpu.async_copy(acc.at[src_v.at[i]], b, gs0).wait()
            pltpu.async_copy(b, part_hbm.at[c, pl.ds(base + i * CW, CW)],
                             gs1)
        for i in range(ROWS_PT // CW - 2, ROWS_PT // CW):
            b = bufs[i % 2]
            pltpu.make_async_copy(
                b, part_hbm.at[c, pl.ds(base + i * CW, CW)], gs1).wait()

    f = pl.kernel(
        body,
        out_type=jax.ShapeDtypeStruct((NC, N_PAD, D), jnp.float32),
        mesh=_MESH,
        scratch_types=[
            pltpu.VMEM((PH, CW), jnp.int32),
            pltpu.VMEM((PH, CW), jnp.int32),
            pltpu.VMEM((CW, D), jnp.float32),
            pltpu.VMEM((CW, D), jnp.float32),
            pltpu.VMEM_SHARED((N_PAD, D), jnp.float32),
            pltpu.SemaphoreType.DMA,
            pltpu.SemaphoreType.DMA,
            pltpu.SemaphoreType.DMA,
        ],
    )
    return f(y, src_t, dst_t)


_CCH = 64


def _sc_combine(part, rin, rout):
    def body(part_hbm, rin_hbm, rout_hbm, x_hbm, y_hbm,
             rin_v, rout_v, a_v, b_v, x_v, y_v, sem):
        w = _worker_id()
        base = w * ROWS_PW
        pltpu.sync_copy(rin_hbm.at[pl.ds(base, ROWS_PW)], rin_v)
        pltpu.sync_copy(rout_hbm.at[pl.ds(base, ROWS_PW)], rout_v)

        def chunk(ch, _):
            off = base + ch * _CCH
            pltpu.sync_copy(part_hbm.at[0, pl.ds(off, _CCH)], a_v)
            pltpu.sync_copy(part_hbm.at[1, pl.ds(off, _CCH)], b_v)

            def row(r, _):
                lr = ch * _CCH + r
                si = rin_v[lr, pl.ds(0, L)]
                so = rout_v[lr, pl.ds(0, L)]
                for j in range(D // L):
                    sl = pl.ds(j * L, L)
                    xv = (a_v[r, sl] + b_v[r, sl]) * si
                    x_v[r, sl] = xv
                    y_v[r, sl] = xv * so
                return 0
            lax.fori_loop(0, _CCH, row, 0)
            pltpu.sync_copy(x_v, x_hbm.at[pl.ds(off, _CCH)])
            pltpu.sync_copy(y_v, y_hbm.at[pl.ds(off, _CCH)])
            return 0
        lax.fori_loop(0, ROWS_PW // _CCH, chunk, 0)

    f = pl.kernel(
        body,
        out_type=(jax.ShapeDtypeStruct((N_PAD, D), jnp.float32),
                  jax.ShapeDtypeStruct((N_PAD, D), jnp.float32)),
        mesh=_MESH,
        scratch_types=[
            pltpu.VMEM((ROWS_PW, L), jnp.float32),
            pltpu.VMEM((ROWS_PW, L), jnp.float32),
            pltpu.VMEM((_CCH, D), jnp.float32),
            pltpu.VMEM((_CCH, D), jnp.float32),
            pltpu.VMEM((_CCH, D), jnp.float32),
            pltpu.VMEM((_CCH, D), jnp.float32),
            pltpu.SemaphoreType.DMA,
        ],
    )
    return f(part, rin, rout)


def _sc_scale0(x0, rout):
    def body(x_hbm, rout_hbm, y_hbm, rout_v, a_v, y_v, sem):
        w = _worker_id()
        base = w * ROWS_PW
        pltpu.sync_copy(rout_hbm.at[pl.ds(base, ROWS_PW)], rout_v)

        def chunk(ch, _):
            off = base + ch * _CCH
            pltpu.sync_copy(x_hbm.at[pl.ds(off, _CCH)], a_v)

            def row(r, _):
                so = rout_v[ch * _CCH + r, pl.ds(0, L)]
                for j in range(D // L):
                    sl = pl.ds(j * L, L)
                    y_v[r, sl] = a_v[r, sl] * so
                return 0
            lax.fori_loop(0, _CCH, row, 0)
            pltpu.sync_copy(y_v, y_hbm.at[pl.ds(off, _CCH)])
            return 0
        lax.fori_loop(0, ROWS_PW // _CCH, chunk, 0)

    f = pl.kernel(
        body,
        out_type=jax.ShapeDtypeStruct((N_PAD, D), jnp.float32),
        mesh=_MESH,
        scratch_types=[
            pltpu.VMEM((ROWS_PW, L), jnp.float32),
            pltpu.VMEM((_CCH, D), jnp.float32),
            pltpu.VMEM((_CCH, D), jnp.float32),
            pltpu.SemaphoreType.DMA,
        ],
    )
    return f(x0, rout)


_MB = 512


def _tc_mlp(nf_p, W1, b1, W2, b2):
    def body(x_ref, w1_ref, b1_ref, w2_ref, b2_ref, o_ref):
        h = jnp.maximum(
            jnp.dot(x_ref[...], w1_ref[...],
                    preferred_element_type=jnp.float32) + b1_ref[...], 0.0)
        o_ref[...] = jnp.dot(h, w2_ref[...],
                             preferred_element_type=jnp.float32) + b2_ref[...]

    return pl.pallas_call(
        body,
        grid=(N_PAD // _MB,),
        in_specs=[
            pl.BlockSpec((_MB, D), lambda i: (i, 0)),
            pl.BlockSpec((D, D), lambda i: (0, 0)),
            pl.BlockSpec((1, D), lambda i: (0, 0)),
            pl.BlockSpec((D, D), lambda i: (0, 0)),
            pl.BlockSpec((1, D), lambda i: (0, 0)),
        ],
        out_specs=pl.BlockSpec((_MB, D), lambda i: (i, 0)),
        out_shape=jax.ShapeDtypeStruct((N_PAD, D), jnp.float32),
    )(nf_p, W1, b1.reshape(1, D), W2, b2.reshape(1, D))


def _tc_rsqrt(deg):
    def body(d_ref, ro_ref, ri_ref):
        ro_ref[...] = lax.rsqrt(jnp.maximum(d_ref[0], 1.0))
        ri_ref[...] = lax.rsqrt(jnp.maximum(d_ref[1], 1.0))

    return pl.pallas_call(
        body,
        out_shape=(jax.ShapeDtypeStruct((N_PAD, L), jnp.float32),
                   jax.ShapeDtypeStruct((N_PAD, L), jnp.float32)),
    )(deg)


_GB = 400


def _tc_gate(xs, gate_W, gate_b):
    n_in = len(xs)

    def body(*refs):
        x_refs = refs[:n_in]
        gw_ref, gb_ref, o_ref = refs[n_in], refs[n_in + 1], refs[n_in + 2]
        gw = gw_ref[...]
        gb = gb_ref[0, 0]
        acc = jnp.zeros((_GB, D), jnp.float32)
        for xr in x_refs:
            xk = xr[...]
            z = jnp.dot(xk, gw, preferred_element_type=jnp.float32) + gb
            acc = acc + xk * jax.nn.sigmoid(z)
        o_ref[...] = acc

    return pl.pallas_call(
        body,
        grid=(N // _GB,),
        in_specs=[pl.BlockSpec((_GB, D), lambda i: (i, 0))] * n_in
        + [pl.BlockSpec((D, 1), lambda i: (0, 0)),
           pl.BlockSpec(memory_space=pltpu.SMEM)],
        out_specs=pl.BlockSpec((_GB, D), lambda i: (i, 0)),
        out_shape=jax.ShapeDtypeStruct((N, D), jnp.float32),
    )(*xs, gate_W, gate_b.reshape(1, 1))


def kernel(node_features, edge_index, W1, b1, W2, b2, gate_W, gate_b,
           is_training=True):
    src = edge_index[0]
    dst = edge_index[1]
    pad = jnp.full((E_PAD - E,), N_PAD - 1, jnp.int32)
    src_f = jnp.concatenate([src, pad])
    dst_f = jnp.concatenate([dst, pad])
    src_t = src_f.reshape(NW, EC, CW)
    dst_t = dst_f.reshape(NW, EC, CW)
    ek_t = jnp.stack([src_f.reshape(NS, ECD, CW),
                      dst_f.reshape(NS, ECD, CW)])

    nf_p = jnp.pad(node_features, ((0, N_PAD - N), (0, 0)))

    x0 = _tc_mlp(nf_p, W1, b1, W2, b2)
    deg = _sc_degrees(ek_t)
    rout, rin = _tc_rsqrt(deg[:, :, :L])

    xs = [x0]
    y = _sc_scale0(x0, rout)
    for _ in range(K):
        part = _sc_round(y, src_t, dst_t)
        xk, y = _sc_combine(part, rin, rout)
        xs.append(xk)

    return _tc_gate(xs, gate_W, gate_b)
